# baseline (speedup 1.0000x reference)
import jax
import jax.numpy as jnp
from jax import lax
from jax.experimental import pallas as pl
from jax.experimental.pallas import tpu as pltpu
from jax.experimental.pallas import tpu_sc as plsc

N = 10000
D = 128
E = 320000
NC = 2
NS = 16

CHUNK = 128
CPT = 80
EP = NC * NS * CPT * CHUNK
EPC = EP // NC
ROWS = EP // CHUNK
CPS = ROWS // NC

ACH = 128
ACPT = 81
AEP = NC * NS * ACPT * ACH
AEPC = AEP // NC

RPT = 624
REM = N - NS * RPT

_mesh = plsc.VectorSubcoreMesh(
    core_axis_name="c", subcore_axis_name="s", num_cores=NC, num_subcores=NS
)


def _deg_body(row1, w1, row2, w2, row3, w3, zn,
              d10, d20, d30, d11, d21, d31, row_t, w_t, semd,
              deg0_s, deg1_s, deg2_s):
    c = lax.axis_index("c")
    s = lax.axis_index("s")

    @pl.when(s == 0)
    def _():
        pltpu.sync_copy(zn, deg0_s)
        pltpu.sync_copy(zn, deg1_s)
        pltpu.sync_copy(zn, deg2_s)

    plsc.subcore_barrier()

    base_r = c * CPS + s * CPT
    for row_h, w_h, deg_s in ((row1, w1, deg0_s), (row2, w2, deg1_s),
                              (row3, w3, deg2_s)):
        pltpu.sync_copy(row_h.at[pl.ds(base_r, CPT)], row_t)
        pltpu.sync_copy(w_h.at[pl.ds(base_r, CPT)], w_t)

        def body(kk, carry, deg_s=deg_s):
            for b in range(10):
                j = kk * 10 + b
                pltpu.async_copy(w_t.at[j], deg_s.at[row_t.at[j]], semd,
                                 add=True)
            for b in range(10):
                pltpu.make_async_copy(w_t.at[0], deg_s.at[row_t.at[0]],
                                      semd).wait()
            return carry
        lax.fori_loop(0, CPT // 10, body, 0)

    plsc.subcore_barrier()

    @pl.when((s == 0) & (c == 0))
    def _():
        pltpu.sync_copy(deg0_s, d10)
        pltpu.sync_copy(deg1_s, d20)
        pltpu.sync_copy(deg2_s, d30)

    @pl.when((s == 0) & (c == 1))
    def _():
        pltpu.sync_copy(deg0_s, d11)
        pltpu.sync_copy(deg1_s, d21)
        pltpu.sync_copy(deg2_s, d31)


_deg_kernel = pl.kernel(
    _deg_body,
    out_type=[jax.ShapeDtypeStruct((N,), jnp.float32)] * 6,
    mesh=_mesh,
    scratch_types=[
        pltpu.VMEM((CPT, CHUNK), jnp.int32),
        pltpu.VMEM((CPT, CHUNK), jnp.float32),
        pltpu.SemaphoreType.DMA,
        pltpu.MemorySpace.VMEM_SHARED((N,), jnp.float32),
        pltpu.MemorySpace.VMEM_SHARED((N,), jnp.float32),
        pltpu.MemorySpace.VMEM_SHARED((N,), jnp.float32),
    ],
)


def _proj_body(x_ref, k1_ref, k2_ref, k3_ref, deg_ref,
               h1_ref, h2_ref, h3_ref, dinv_ref):
    deg = deg_ref[0, :, :, 0] + deg_ref[1, :, :, 0] + 1.0
    safe = jnp.where(deg > 0, deg, 1.0)
    dinv = jnp.where(deg > 0, lax.rsqrt(safe), 0.0)
    dinv_ref[...] = dinv[:, :, None]
    x = x_ref[...]
    for p, (k_ref, h_ref) in enumerate(((k1_ref, h1_ref), (k2_ref, h2_ref),
                                        (k3_ref, h3_ref))):
        h = jnp.dot(x, k_ref[...], preferred_element_type=jnp.float32)
        h_ref[...] = dinv[p][:, None] * h


_BLK = 1000


def _proj(x, k1, k2, k3, deg_part):
    grid = N // _BLK
    deg4 = deg_part[:, :, :, None]
    return pl.pallas_call(
        _proj_body,
        grid=(grid,),
        in_specs=[
            pl.BlockSpec((_BLK, D), lambda i: (i, 0)),
            pl.BlockSpec((D, D), lambda i: (0, 0)),
            pl.BlockSpec((D, D), lambda i: (0, 0)),
            pl.BlockSpec((D, D), lambda i: (0, 0)),
            pl.BlockSpec((NC, 3, _BLK, 1), lambda i: (0, 0, i, 0)),
        ],
        out_specs=[
            pl.BlockSpec((_BLK, D), lambda i: (i, 0)),
            pl.BlockSpec((_BLK, D), lambda i: (i, 0)),
            pl.BlockSpec((_BLK, D), lambda i: (i, 0)),
            pl.BlockSpec((3, _BLK, 1), lambda i: (0, i, 0)),
        ],
        out_shape=[
            jax.ShapeDtypeStruct((N, D), jnp.float32),
            jax.ShapeDtypeStruct((N, D), jnp.float32),
            jax.ShapeDtypeStruct((N, D), jnp.float32),
            jax.ShapeDtypeStruct((3, N, 1), jnp.float32),
        ],
    )(x, k1, k2, k3, deg4)


def _agg_body(h1, col1, row1, w1, h2, col2, row2, w2, h3, col3, row3, w3,
              zf, out1, out2, out3,
              b0, b1, b2, ic0, ic1, ic2, ir0, ir1, ir2, iw0, iw1, iw2,
              sg0, sg1, sg2, ss0, ss1, ss2, scw0, scw1, scw2,
              sr0, sr1, sr2, acc_s):
    c = lax.axis_index("c")
    s = lax.axis_index("s")
    bufs = (b0, b1, b2)
    cols = (ic0, ic1, ic2)
    rows = (ir0, ir1, ir2)
    ws = (iw0, iw1, iw2)
    sgs = (sg0, sg1, sg2)
    sss = (ss0, ss1, ss2)
    scws = (scw0, scw1, scw2)
    srs = (sr0, sr1, sr2)
    ebase = c * AEPC + s * (ACPT * ACH)

    for h_h, col_h, row_h, w_h, out_h in (
        (h1, col1, row1, w1, out1),
        (h2, col2, row2, w2, out2),
        (h3, col3, row3, w3, out3),
    ):
        @pl.when(c == 0)
        def _(h_h=h_h):
            pltpu.sync_copy(h_h.at[pl.ds(s * RPT, RPT)],
                            acc_s.at[pl.ds(s * RPT, RPT)])

            @pl.when(s == 0)
            def _():
                pltpu.sync_copy(h_h.at[pl.ds(NS * RPT, REM)],
                                acc_s.at[pl.ds(NS * RPT, REM)])

        @pl.when(c != 0)
        def _():
            pltpu.sync_copy(zf.at[pl.ds(s * RPT, RPT)],
                            acc_s.at[pl.ds(s * RPT, RPT)])

            @pl.when(s == 0)
            def _():
                pltpu.sync_copy(zf.at[pl.ds(NS * RPT, REM)],
                                acc_s.at[pl.ds(NS * RPT, REM)])

        plsc.subcore_barrier()

        def scale(gbuf, sbuf, wbuf):
            def group(g, carry):
                w16 = wbuf[pl.ds(g * 16, 16)]
                for l in range(16):
                    wv = w16[l]
                    e = g * 16 + l
                    for q in range(D // 16):
                        sbuf[e, pl.ds(q * 16, 16)] = (
                            gbuf[e, pl.ds(q * 16, 16)] * wv)
                return carry
            lax.fori_loop(0, ACH // 16, group, 0)

        def issue_colw(k, q, col_h=col_h, w_h=w_h):
            base = ebase + k * ACH
            pltpu.async_copy(col_h.at[pl.ds(base, ACH)], cols[q], scws[q])
            pltpu.async_copy(w_h.at[pl.ds(base, ACH)], ws[q], scws[q])

        def wait_colw(q, col_h=col_h, w_h=w_h):
            pltpu.make_async_copy(col_h.at[pl.ds(0, ACH)], cols[q],
                                  scws[q]).wait()
            pltpu.make_async_copy(w_h.at[pl.ds(0, ACH)], ws[q],
                                  scws[q]).wait()

        def issue_rows(k, q, row_h=row_h):
            base = ebase + k * ACH
            pltpu.async_copy(row_h.at[pl.ds(base, ACH)], rows[q], srs[q])

        def wait_rows(q, row_h=row_h):
            pltpu.make_async_copy(row_h.at[pl.ds(0, ACH)], rows[q],
                                  srs[q]).wait()

        issue_colw(0, 0)
        issue_colw(1, 1)
        issue_rows(0, 0)
        wait_colw(0)
        pltpu.async_copy(h_h.at[cols[0]], b0, sg0)

        def body(kk, carry, h_h=h_h, col_h=col_h, row_h=row_h, w_h=w_h):
            for u in range(3):
                j = kk * 3 + u
                b = u % 3
                n1 = (u + 1) % 3
                n2 = (u + 2) % 3

                @pl.when(j >= 2)
                def _(n1=n1):
                    pltpu.make_async_copy(bufs[n1],
                                          acc_s.at[pl.ds(0, ACH)],
                                          sss[n1]).wait()

                @pl.when(j + 2 < ACPT)
                def _(j=j, n2=n2):
                    issue_colw(j + 2, n2)

                @pl.when(j + 1 < ACPT)
                def _(j=j, n1=n1):
                    issue_rows(j + 1, n1)

                pltpu.make_async_copy(h_h.at[pl.ds(0, ACH)], bufs[b],
                                      sgs[b]).wait()
                scale(bufs[b], bufs[b], ws[b])
                wait_rows(b)
                pltpu.async_copy(bufs[b], acc_s.at[rows[b]], sss[b],
                                 add=True)

                @pl.when(j + 1 < ACPT)
                def _(n1=n1, h_h=h_h):
                    wait_colw(n1)
                    pltpu.async_copy(h_h.at[cols[n1]], bufs[n1], sgs[n1])
            return carry

        lax.fori_loop(0, ACPT // 3, body, 0)

        pltpu.make_async_copy(bufs[(ACPT - 2) % 3],
                              acc_s.at[pl.ds(0, ACH)],
                              sss[(ACPT - 2) % 3]).wait()
        pltpu.make_async_copy(bufs[(ACPT - 1) % 3],
                              acc_s.at[pl.ds(0, ACH)],
                              sss[(ACPT - 1) % 3]).wait()

        plsc.subcore_barrier()
        pltpu.sync_copy(acc_s.at[pl.ds(s * RPT, RPT)],
                        out_h.at[c, pl.ds(s * RPT, RPT)])

        @pl.when(s == 0)
        def _(out_h=out_h):
            pltpu.sync_copy(acc_s.at[pl.ds(NS * RPT, REM)],
                            out_h.at[c, pl.ds(NS * RPT, REM)])

        plsc.subcore_barrier()


_agg_kernel = pl.kernel(
    _agg_body,
    out_type=[
        jax.ShapeDtypeStruct((NC, N, D), jnp.float32),
        jax.ShapeDtypeStruct((NC, N, D), jnp.float32),
        jax.ShapeDtypeStruct((NC, N, D), jnp.float32),
    ],
    mesh=_mesh,
    scratch_types=(
        [pltpu.VMEM((ACH, D), jnp.float32)] * 3
        + [pltpu.VMEM((ACH,), jnp.int32)] * 6
        + [pltpu.VMEM((ACH,), jnp.float32)] * 3
        + [pltpu.SemaphoreType.DMA] * 12
        + [pltpu.MemorySpace.VMEM_SHARED((N, D), jnp.float32)]
    ),
)


def _final_body(a1_ref, a2_ref, a3_ref, dinv_ref, b1_ref, b2_ref, b3_ref,
                fc_ref, fcb_ref, cw_ref, out_ref):
    dinv = dinv_ref[...]
    ys = []
    for p, (a_ref, b_ref) in enumerate(((a1_ref, b1_ref), (a2_ref, b2_ref),
                                        (a3_ref, b3_ref))):
        a = a_ref[0] + a_ref[1]
        y = jnp.maximum(dinv[p] * a + b_ref[...], 0.0)
        ys.append(y)
    t = jnp.dot(ys[0], fc_ref[0:D, :], preferred_element_type=jnp.float32)
    t = t + cw_ref[0] * jnp.dot(ys[1], fc_ref[D:2 * D, :],
                                preferred_element_type=jnp.float32)
    t = t + cw_ref[1] * jnp.dot(ys[2], fc_ref[2 * D:3 * D, :],
                                preferred_element_type=jnp.float32)
    out_ref[...] = jnp.maximum(t + fcb_ref[...], 0.0)


def _final(acc1, acc2, acc3, dinv, b1, b2, b3, fck, fcb, cw):
    grid = N // _BLK
    return pl.pallas_call(
        _final_body,
        grid=(grid,),
        in_specs=[
            pl.BlockSpec((NC, _BLK, D), lambda i: (0, i, 0)),
            pl.BlockSpec((NC, _BLK, D), lambda i: (0, i, 0)),
            pl.BlockSpec((NC, _BLK, D), lambda i: (0, i, 0)),
            pl.BlockSpec((3, _BLK, 1), lambda i: (0, i, 0)),
            pl.BlockSpec((1, D), lambda i: (0, 0)),
            pl.BlockSpec((1, D), lambda i: (0, 0)),
            pl.BlockSpec((1, D), lambda i: (0, 0)),
            pl.BlockSpec((3 * D, D), lambda i: (0, 0)),
            pl.BlockSpec((1, D), lambda i: (0, 0)),
            pl.BlockSpec(memory_space=pltpu.MemorySpace.SMEM),
        ],
        out_specs=pl.BlockSpec((_BLK, D), lambda i: (i, 0)),
        out_shape=jax.ShapeDtypeStruct((N, D), jnp.float32),
    )(acc1, acc2, acc3, dinv, b1.reshape(1, D), b2.reshape(1, D),
      b3.reshape(1, D), fck, fcb.reshape(1, D), cw)


def _pad_edges(edge_index, edge_weight):
    row = edge_index[0].astype(jnp.int32)
    col = edge_index[1].astype(jnp.int32)
    w = edge_weight.astype(jnp.float32)

    pad_d = EP - E
    pidx_d = jnp.arange(pad_d, dtype=jnp.int32) % N
    row2d = jnp.concatenate([row, pidx_d]).reshape(ROWS, CHUNK)
    w2d = jnp.concatenate([w, jnp.zeros((pad_d,), jnp.float32)]
                          ).reshape(ROWS, CHUNK)

    pad_a = AEP - E
    pidx_a = jnp.arange(pad_a, dtype=jnp.int32) % N
    rowf = jnp.concatenate([row, pidx_a])
    colf = jnp.concatenate([col, pidx_a])
    wf = jnp.concatenate([w, jnp.zeros((pad_a,), jnp.float32)])
    return row2d, w2d, rowf, colf, wf


def kernel(x, edge_index_1st_prox, edge_weight_1st_prox,
           edge_index_2nd_prox_in, edge_weight_2nd_prox_in,
           edge_index_2nd_prox_out, edge_weight_2nd_prox_out,
           kernel1, kernel2in, kernel2out, bias1, bias2in, bias2out,
           concate_weight2in, concate_weight2out, fc_kernel, fc_bias):
    r1d, w1d, row1, col1, w1 = _pad_edges(edge_index_1st_prox,
                                          edge_weight_1st_prox)
    r2d, w2d, row2, col2, w2 = _pad_edges(edge_index_2nd_prox_in,
                                          edge_weight_2nd_prox_in)
    r3d, w3d, row3, col3, w3 = _pad_edges(edge_index_2nd_prox_out,
                                          edge_weight_2nd_prox_out)

    zn = jnp.zeros((N,), jnp.float32)
    zf = jnp.zeros((N, D), jnp.float32)

    d10, d20, d30, d11, d21, d31 = _deg_kernel(r1d, w1d, r2d, w2d,
                                               r3d, w3d, zn)
    deg_part = jnp.stack([jnp.stack([d10, d20, d30]),
                          jnp.stack([d11, d21, d31])])
    h1, h2, h3, dinv = _proj(x, kernel1, kernel2in, kernel2out, deg_part)
    acc1, acc2, acc3 = _agg_kernel(h1, col1, row1, w1, h2, col2, row2, w2,
                                   h3, col3, row3, w3, zf)
    cw = jnp.concatenate([concate_weight2in, concate_weight2out])
    return _final(acc1, acc2, acc3, dinv, bias1, bias2in, bias2out,
                  fc_kernel, fc_bias, cw)


# speedup vs baseline: 1.1763x; 1.1763x over previous
import jax
import jax.numpy as jnp
from jax import lax
from jax.experimental import pallas as pl
from jax.experimental.pallas import tpu as pltpu
from jax.experimental.pallas import tpu_sc as plsc

N = 10000
D = 128
E = 320000
NC = 2
NS = 16
CHUNK = 128
EP = 327680
ROWS = EP // CHUNK
CPT = ROWS // (NC * NS)
CPS = ROWS // NC
ACH = 64
ATPC = EP // (NC * NS * ACH)
ABATCH = 32
AROWS = EP // ACH
AROWSC = AROWS // NC
RPT = 624
REM = N - NS * RPT

_mesh = plsc.VectorSubcoreMesh(
    core_axis_name="c", subcore_axis_name="s", num_cores=NC, num_subcores=NS
)


def _deg_body(row1, w1, row2, w2, row3, w3, zn,
              d10, d20, d30, d11, d21, d31, row_t, w_t, semd,
              deg0_s, deg1_s, deg2_s):
    c = lax.axis_index("c")
    s = lax.axis_index("s")

    @pl.when(s == 0)
    def _():
        pltpu.sync_copy(zn, deg0_s)
        pltpu.sync_copy(zn, deg1_s)
        pltpu.sync_copy(zn, deg2_s)

    plsc.subcore_barrier()

    base_r = c * CPS + s * CPT
    for row_h, w_h, deg_s in ((row1, w1, deg0_s), (row2, w2, deg1_s),
                              (row3, w3, deg2_s)):
        pltpu.sync_copy(row_h.at[pl.ds(base_r, CPT)], row_t)
        pltpu.sync_copy(w_h.at[pl.ds(base_r, CPT)], w_t)

        def body(kk, carry, deg_s=deg_s):
            for b in range(10):
                j = kk * 10 + b
                pltpu.async_copy(w_t.at[j], deg_s.at[row_t.at[j]], semd,
                                 add=True)
            for b in range(10):
                pltpu.make_async_copy(w_t.at[0], deg_s.at[row_t.at[0]],
                                      semd).wait()
            return carry
        lax.fori_loop(0, CPT // 10, body, 0)

    plsc.subcore_barrier()

    @pl.when((s == 0) & (c == 0))
    def _():
        pltpu.sync_copy(deg0_s, d10)
        pltpu.sync_copy(deg1_s, d20)
        pltpu.sync_copy(deg2_s, d30)

    @pl.when((s == 0) & (c == 1))
    def _():
        pltpu.sync_copy(deg0_s, d11)
        pltpu.sync_copy(deg1_s, d21)
        pltpu.sync_copy(deg2_s, d31)


_deg_kernel = pl.kernel(
    _deg_body,
    out_type=[jax.ShapeDtypeStruct((N,), jnp.float32)] * 6,
    mesh=_mesh,
    scratch_types=[
        pltpu.VMEM((CPT, CHUNK), jnp.int32),
        pltpu.VMEM((CPT, CHUNK), jnp.float32),
        pltpu.SemaphoreType.DMA,
        pltpu.MemorySpace.VMEM_SHARED((N,), jnp.float32),
        pltpu.MemorySpace.VMEM_SHARED((N,), jnp.float32),
        pltpu.MemorySpace.VMEM_SHARED((N,), jnp.float32),
    ],
)


def _proj_body(x_ref, k1_ref, k2_ref, k3_ref, deg_ref,
               h1_ref, h2_ref, h3_ref, dinv_ref):
    deg = deg_ref[0, :, :, 0] + deg_ref[1, :, :, 0] + 1.0
    safe = jnp.where(deg > 0, deg, 1.0)
    dinv = jnp.where(deg > 0, lax.rsqrt(safe), 0.0)
    dinv_ref[...] = dinv[:, :, None]
    x = x_ref[...]
    for p, (k_ref, h_ref) in enumerate(((k1_ref, h1_ref), (k2_ref, h2_ref),
                                        (k3_ref, h3_ref))):
        h = jnp.dot(x, k_ref[...], preferred_element_type=jnp.float32)
        h_ref[...] = dinv[p][:, None] * h


_BLK = 1000


def _proj(x, k1, k2, k3, deg_part):
    grid = N // _BLK
    deg4 = deg_part[:, :, :, None]
    return pl.pallas_call(
        _proj_body,
        grid=(grid,),
        in_specs=[
            pl.BlockSpec((_BLK, D), lambda i: (i, 0)),
            pl.BlockSpec((D, D), lambda i: (0, 0)),
            pl.BlockSpec((D, D), lambda i: (0, 0)),
            pl.BlockSpec((D, D), lambda i: (0, 0)),
            pl.BlockSpec((NC, 3, _BLK, 1), lambda i: (0, 0, i, 0)),
        ],
        out_specs=[
            pl.BlockSpec((_BLK, D), lambda i: (i, 0)),
            pl.BlockSpec((_BLK, D), lambda i: (i, 0)),
            pl.BlockSpec((_BLK, D), lambda i: (i, 0)),
            pl.BlockSpec((3, _BLK, 1), lambda i: (0, i, 0)),
        ],
        out_shape=[
            jax.ShapeDtypeStruct((N, D), jnp.float32),
            jax.ShapeDtypeStruct((N, D), jnp.float32),
            jax.ShapeDtypeStruct((N, D), jnp.float32),
            jax.ShapeDtypeStruct((3, N, 1), jnp.float32),
        ],
    )(x, k1, k2, k3, deg4)


def _agg_body(h1, col1, row1, w1, h2, col2, row2, w2, h3, col3, row3, w3,
              zf, out1, out2, out3,
              b0, b1, b2, b3, col_t, row_t, w_t,
              sg0, sg1, sg2, sg3, ss0, ss1, ss2, ss3, acc_s):
    c = lax.axis_index("c")
    s = lax.axis_index("s")
    bufs = (b0, b1, b2, b3)
    sgs = (sg0, sg1, sg2, sg3)
    sss = (ss0, ss1, ss2, ss3)
    base_r = c * AROWSC + s * ATPC

    for h_h, col_h, row_h, w_h, out_h in (
        (h1, col1, row1, w1, out1),
        (h2, col2, row2, w2, out2),
        (h3, col3, row3, w3, out3),
    ):
        @pl.when(c == 0)
        def _(h_h=h_h):
            pltpu.sync_copy(h_h.at[pl.ds(s * RPT, RPT)],
                            acc_s.at[pl.ds(s * RPT, RPT)])

            @pl.when(s == 0)
            def _():
                pltpu.sync_copy(h_h.at[pl.ds(NS * RPT, REM)],
                                acc_s.at[pl.ds(NS * RPT, REM)])

        @pl.when(c != 0)
        def _():
            pltpu.sync_copy(zf.at[pl.ds(s * RPT, RPT)],
                            acc_s.at[pl.ds(s * RPT, RPT)])

            @pl.when(s == 0)
            def _():
                pltpu.sync_copy(zf.at[pl.ds(NS * RPT, REM)],
                                acc_s.at[pl.ds(NS * RPT, REM)])

        plsc.subcore_barrier()

        def scale(buf, k):
            def group(g, carry):
                w16 = w_t[k, pl.ds(g * 16, 16)]
                for l in range(16):
                    wv = w16[l]
                    e = g * 16 + l
                    for q in range(D // 16):
                        buf[e, pl.ds(q * 16, 16)] = (
                            buf[e, pl.ds(q * 16, 16)] * wv)
                return carry
            lax.fori_loop(0, ACH // 16, group, 0)

        def batch_body(batch, bcarry, h_h=h_h, col_h=col_h, row_h=row_h,
                       w_h=w_h):
            bbase = base_r + batch * ABATCH
            pltpu.sync_copy(col_h.at[pl.ds(bbase, ABATCH)], col_t)
            pltpu.sync_copy(row_h.at[pl.ds(bbase, ABATCH)], row_t)
            pltpu.sync_copy(w_h.at[pl.ds(bbase, ABATCH)], w_t)

            pltpu.async_copy(h_h.at[col_t.at[0]], b0, sg0)
            pltpu.async_copy(h_h.at[col_t.at[1]], b1, sg1)

            def body(kk, carry, h_h=h_h):
                for u in range(4):
                    k = kk * 4 + u
                    n2 = (u + 2) % 4

                    @pl.when(k >= 2)
                    def _(n2=n2):
                        pltpu.make_async_copy(bufs[n2],
                                              acc_s.at[pl.ds(0, ACH)],
                                              sss[n2]).wait()

                    pltpu.make_async_copy(h_h.at[pl.ds(0, ACH)],
                                          bufs[u], sgs[u]).wait()
                    scale(bufs[u], k)
                    pltpu.async_copy(bufs[u], acc_s.at[row_t.at[k]],
                                     sss[u], add=True)

                    @pl.when(k + 2 < ABATCH)
                    def _(k=k, n2=n2, h_h=h_h):
                        pltpu.async_copy(h_h.at[col_t.at[k + 2]],
                                         bufs[n2], sgs[n2])
                return carry

            lax.fori_loop(0, ABATCH // 4, body, 0)

            pltpu.make_async_copy(bufs[(ABATCH - 2) % 4],
                                  acc_s.at[pl.ds(0, ACH)],
                                  sss[(ABATCH - 2) % 4]).wait()
            pltpu.make_async_copy(bufs[(ABATCH - 1) % 4],
                                  acc_s.at[pl.ds(0, ACH)],
                                  sss[(ABATCH - 1) % 4]).wait()
            return bcarry

        lax.fori_loop(0, ATPC // ABATCH, batch_body, 0)

        plsc.subcore_barrier()
        pltpu.sync_copy(acc_s.at[pl.ds(s * RPT, RPT)],
                        out_h.at[c, pl.ds(s * RPT, RPT)])

        @pl.when(s == 0)
        def _(out_h=out_h):
            pltpu.sync_copy(acc_s.at[pl.ds(NS * RPT, REM)],
                            out_h.at[c, pl.ds(NS * RPT, REM)])

        plsc.subcore_barrier()


_agg_kernel = pl.kernel(
    _agg_body,
    out_type=[
        jax.ShapeDtypeStruct((NC, N, D), jnp.float32),
        jax.ShapeDtypeStruct((NC, N, D), jnp.float32),
        jax.ShapeDtypeStruct((NC, N, D), jnp.float32),
    ],
    mesh=_mesh,
    scratch_types=(
        [pltpu.VMEM((ACH, D), jnp.float32)] * 4
        + [pltpu.VMEM((ABATCH, ACH), jnp.int32),
           pltpu.VMEM((ABATCH, ACH), jnp.int32),
           pltpu.VMEM((ABATCH, ACH), jnp.float32)]
        + [pltpu.SemaphoreType.DMA] * 8
        + [pltpu.MemorySpace.VMEM_SHARED((N, D), jnp.float32)]
    ),
)


def _final_body(a1_ref, a2_ref, a3_ref, dinv_ref, b1_ref, b2_ref, b3_ref,
                fc_ref, fcb_ref, cw_ref, out_ref):
    dinv = dinv_ref[...]
    ys = []
    for p, (a_ref, b_ref) in enumerate(((a1_ref, b1_ref), (a2_ref, b2_ref),
                                        (a3_ref, b3_ref))):
        a = a_ref[0] + a_ref[1]
        y = jnp.maximum(dinv[p] * a + b_ref[...], 0.0)
        ys.append(y)
    t = jnp.dot(ys[0], fc_ref[0:D, :], preferred_element_type=jnp.float32)
    t = t + cw_ref[0] * jnp.dot(ys[1], fc_ref[D:2 * D, :],
                                preferred_element_type=jnp.float32)
    t = t + cw_ref[1] * jnp.dot(ys[2], fc_ref[2 * D:3 * D, :],
                                preferred_element_type=jnp.float32)
    out_ref[...] = jnp.maximum(t + fcb_ref[...], 0.0)


def _final(acc1, acc2, acc3, dinv, b1, b2, b3, fck, fcb, cw):
    grid = N // _BLK
    return pl.pallas_call(
        _final_body,
        grid=(grid,),
        in_specs=[
            pl.BlockSpec((NC, _BLK, D), lambda i: (0, i, 0)),
            pl.BlockSpec((NC, _BLK, D), lambda i: (0, i, 0)),
            pl.BlockSpec((NC, _BLK, D), lambda i: (0, i, 0)),
            pl.BlockSpec((3, _BLK, 1), lambda i: (0, i, 0)),
            pl.BlockSpec((1, D), lambda i: (0, 0)),
            pl.BlockSpec((1, D), lambda i: (0, 0)),
            pl.BlockSpec((1, D), lambda i: (0, 0)),
            pl.BlockSpec((3 * D, D), lambda i: (0, 0)),
            pl.BlockSpec((1, D), lambda i: (0, 0)),
            pl.BlockSpec(memory_space=pltpu.MemorySpace.SMEM),
        ],
        out_specs=pl.BlockSpec((_BLK, D), lambda i: (i, 0)),
        out_shape=jax.ShapeDtypeStruct((N, D), jnp.float32),
    )(acc1, acc2, acc3, dinv, b1.reshape(1, D), b2.reshape(1, D),
      b3.reshape(1, D), fck, fcb.reshape(1, D), cw)


def _pad_edges(edge_index, edge_weight):
    row = edge_index[0].astype(jnp.int32)
    col = edge_index[1].astype(jnp.int32)
    w = edge_weight.astype(jnp.float32)
    pad = EP - E
    pidx = jnp.arange(pad, dtype=jnp.int32) % N
    rowf = jnp.concatenate([row, pidx])
    colf = jnp.concatenate([col, pidx])
    wf = jnp.concatenate([w, jnp.zeros((pad,), jnp.float32)])
    return (rowf.reshape(ROWS, CHUNK), wf.reshape(ROWS, CHUNK),
            rowf.reshape(AROWS, ACH), colf.reshape(AROWS, ACH),
            wf.reshape(AROWS, ACH))


def kernel(x, edge_index_1st_prox, edge_weight_1st_prox,
           edge_index_2nd_prox_in, edge_weight_2nd_prox_in,
           edge_index_2nd_prox_out, edge_weight_2nd_prox_out,
           kernel1, kernel2in, kernel2out, bias1, bias2in, bias2out,
           concate_weight2in, concate_weight2out, fc_kernel, fc_bias):
    r1d, w1d, row1, col1, w1 = _pad_edges(edge_index_1st_prox,
                                          edge_weight_1st_prox)
    r2d, w2d, row2, col2, w2 = _pad_edges(edge_index_2nd_prox_in,
                                          edge_weight_2nd_prox_in)
    r3d, w3d, row3, col3, w3 = _pad_edges(edge_index_2nd_prox_out,
                                          edge_weight_2nd_prox_out)

    zn = jnp.zeros((N,), jnp.float32)
    zf = jnp.zeros((N, D), jnp.float32)

    d10, d20, d30, d11, d21, d31 = _deg_kernel(r1d, w1d, r2d, w2d,
                                               r3d, w3d, zn)
    deg_part = jnp.stack([jnp.stack([d10, d20, d30]),
                          jnp.stack([d11, d21, d31])])
    h1, h2, h3, dinv = _proj(x, kernel1, kernel2in, kernel2out, deg_part)
    acc1, acc2, acc3 = _agg_kernel(h1, col1, row1, w1, h2, col2, row2, w2,
                                   h3, col3, row3, w3, zf)
    cw = jnp.concatenate([concate_weight2in, concate_weight2out])
    return _final(acc1, acc2, acc3, dinv, bias1, bias2in, bias2out,
                  fc_kernel, fc_bias, cw)


# speedup vs baseline: 1.2310x; 1.0465x over previous
import jax
import jax.numpy as jnp
from jax import lax
from jax.experimental import pallas as pl
from jax.experimental.pallas import tpu as pltpu
from jax.experimental.pallas import tpu_sc as plsc

N = 10000
D = 128
E = 320000
NC = 2
NS = 16
CHUNK = 128
CPT = 80
EP = NC * NS * CPT * CHUNK
EPC = EP // NC
ROWS = EP // CHUNK
CPS = ROWS // NC
RPT = 624
REM = N - NS * RPT
NBUF = 2
HCPT = CPT // 2

_mesh = plsc.VectorSubcoreMesh(
    core_axis_name="c", subcore_axis_name="s", num_cores=NC, num_subcores=NS
)


def _deg_body(row1, w1, row2, w2, row3, w3, zn,
              d10, d20, d30, d11, d21, d31, row_t, w_t, semd,
              deg0_s, deg1_s, deg2_s):
    c = lax.axis_index("c")
    s = lax.axis_index("s")

    @pl.when(s == 0)
    def _():
        pltpu.sync_copy(zn, deg0_s)
        pltpu.sync_copy(zn, deg1_s)
        pltpu.sync_copy(zn, deg2_s)

    plsc.subcore_barrier()

    base_r = c * CPS + s * CPT
    for row_h, w_h, deg_s in ((row1, w1, deg0_s), (row2, w2, deg1_s),
                              (row3, w3, deg2_s)):
        pltpu.sync_copy(row_h.at[pl.ds(base_r, CPT)], row_t)
        pltpu.sync_copy(w_h.at[pl.ds(base_r, CPT)], w_t)

        def body(kk, carry, deg_s=deg_s):
            for b in range(10):
                j = kk * 10 + b
                pltpu.async_copy(w_t.at[j], deg_s.at[row_t.at[j]], semd,
                                 add=True)
            for b in range(10):
                pltpu.make_async_copy(w_t.at[0], deg_s.at[row_t.at[0]],
                                      semd).wait()
            return carry
        lax.fori_loop(0, CPT // 10, body, 0)

    plsc.subcore_barrier()

    @pl.when((s == 0) & (c == 0))
    def _():
        pltpu.sync_copy(deg0_s, d10)
        pltpu.sync_copy(deg1_s, d20)
        pltpu.sync_copy(deg2_s, d30)

    @pl.when((s == 0) & (c == 1))
    def _():
        pltpu.sync_copy(deg0_s, d11)
        pltpu.sync_copy(deg1_s, d21)
        pltpu.sync_copy(deg2_s, d31)


_deg_kernel = pl.kernel(
    _deg_body,
    out_type=[jax.ShapeDtypeStruct((N,), jnp.float32)] * 6,
    mesh=_mesh,
    scratch_types=[
        pltpu.VMEM((CPT, CHUNK), jnp.int32),
        pltpu.VMEM((CPT, CHUNK), jnp.float32),
        pltpu.SemaphoreType.DMA,
        pltpu.MemorySpace.VMEM_SHARED((N,), jnp.float32),
        pltpu.MemorySpace.VMEM_SHARED((N,), jnp.float32),
        pltpu.MemorySpace.VMEM_SHARED((N,), jnp.float32),
    ],
)


def _proj_body(x_ref, k1_ref, k2_ref, k3_ref, deg_ref,
               h1_ref, h2_ref, h3_ref, dinv_ref):
    deg = deg_ref[0, :, :, 0] + deg_ref[1, :, :, 0] + 1.0
    safe = jnp.where(deg > 0, deg, 1.0)
    dinv = jnp.where(deg > 0, lax.rsqrt(safe), 0.0)
    dinv_ref[...] = dinv[:, :, None]
    x = x_ref[...]
    for p, (k_ref, h_ref) in enumerate(((k1_ref, h1_ref), (k2_ref, h2_ref),
                                        (k3_ref, h3_ref))):
        h = jnp.dot(x, k_ref[...], preferred_element_type=jnp.float32)
        h_ref[...] = dinv[p][:, None] * h


_BLK = 1000


def _proj(x, k1, k2, k3, deg_part):
    grid = N // _BLK
    deg4 = deg_part[:, :, :, None]
    return pl.pallas_call(
        _proj_body,
        grid=(grid,),
        in_specs=[
            pl.BlockSpec((_BLK, D), lambda i: (i, 0)),
            pl.BlockSpec((D, D), lambda i: (0, 0)),
            pl.BlockSpec((D, D), lambda i: (0, 0)),
            pl.BlockSpec((D, D), lambda i: (0, 0)),
            pl.BlockSpec((NC, 3, _BLK, 1), lambda i: (0, 0, i, 0)),
        ],
        out_specs=[
            pl.BlockSpec((_BLK, D), lambda i: (i, 0)),
            pl.BlockSpec((_BLK, D), lambda i: (i, 0)),
            pl.BlockSpec((_BLK, D), lambda i: (i, 0)),
            pl.BlockSpec((3, _BLK, 1), lambda i: (0, i, 0)),
        ],
        out_shape=[
            jax.ShapeDtypeStruct((N, D), jnp.float32),
            jax.ShapeDtypeStruct((N, D), jnp.float32),
            jax.ShapeDtypeStruct((N, D), jnp.float32),
            jax.ShapeDtypeStruct((3, N, 1), jnp.float32),
        ],
    )(x, k1, k2, k3, deg4)


def _agg_body(h1, col1, row1, w1, h2, col2, row2, w2, h3, col3, row3, w3,
              zf, out1, out2, out3,
              col_t, row_t, w_t, rows0, rows1,
              sg0, sg1, ss0, ss1, acc_s):
    c = lax.axis_index("c")
    s = lax.axis_index("s")
    bufs = (rows0, rows1)
    sgs = (sg0, sg1)
    sss = (ss0, ss1)
    base_r = c * CPS + s * CPT

    for h_h, col_h, row_h, w_h, out_h in (
        (h1, col1, row1, w1, out1),
        (h2, col2, row2, w2, out2),
        (h3, col3, row3, w3, out3),
    ):
        @pl.when(c == 0)
        def _(h_h=h_h):
            pltpu.sync_copy(h_h.at[pl.ds(s * RPT, RPT)],
                            acc_s.at[pl.ds(s * RPT, RPT)])

            @pl.when(s == 0)
            def _():
                pltpu.sync_copy(h_h.at[pl.ds(NS * RPT, REM)],
                                acc_s.at[pl.ds(NS * RPT, REM)])

        @pl.when(c != 0)
        def _():
            pltpu.sync_copy(zf.at[pl.ds(s * RPT, RPT)],
                            acc_s.at[pl.ds(s * RPT, RPT)])

            @pl.when(s == 0)
            def _():
                pltpu.sync_copy(zf.at[pl.ds(NS * RPT, REM)],
                                acc_s.at[pl.ds(NS * RPT, REM)])

        plsc.subcore_barrier()

        def scale(buf, k):
            def group(g, carry):
                w16 = w_t[k, pl.ds(g * 16, 16)]
                for l in range(16):
                    wv = w16[l]
                    e = g * 16 + l
                    for q in range(D // 16):
                        buf[e, pl.ds(q * 16, 16)] = (
                            buf[e, pl.ds(q * 16, 16)] * wv)
                return carry
            lax.fori_loop(0, CHUNK // 16, group, 0)

        for half in range(2):
            hbase = base_r + half * HCPT
            pltpu.sync_copy(col_h.at[pl.ds(hbase, HCPT)], col_t)
            pltpu.sync_copy(row_h.at[pl.ds(hbase, HCPT)], row_t)
            pltpu.sync_copy(w_h.at[pl.ds(hbase, HCPT)], w_t)

            pltpu.async_copy(h_h.at[col_t.at[0]], bufs[0], sgs[0])

            def body(kk, carry, h_h=h_h):
                for b in range(NBUF):
                    k = kk * NBUF + b
                    nb = (b + NBUF - 1) % NBUF

                    @pl.when(k + NBUF - 1 < HCPT)
                    def _(k=k, nb=nb, h_h=h_h):
                        @pl.when(k >= 1)
                        def _():
                            pltpu.make_async_copy(
                                bufs[nb], acc_s.at[row_t.at[0]],
                                sss[nb]).wait()

                        pltpu.async_copy(h_h.at[col_t.at[k + NBUF - 1]],
                                         bufs[nb], sgs[nb])

                    pltpu.make_async_copy(h_h.at[col_t.at[k]], bufs[b],
                                          sgs[b]).wait()
                    scale(bufs[b], k)
                    pltpu.async_copy(bufs[b], acc_s.at[row_t.at[k]],
                                     sss[b], add=True)
                return carry

            lax.fori_loop(0, HCPT // NBUF, body, 0)

            for j in range(HCPT - NBUF, HCPT):
                b = j % NBUF
                pltpu.make_async_copy(bufs[b], acc_s.at[row_t.at[0]],
                                      sss[b]).wait()

        plsc.subcore_barrier()
        pltpu.sync_copy(acc_s.at[pl.ds(s * RPT, RPT)],
                        out_h.at[c, pl.ds(s * RPT, RPT)])

        @pl.when(s == 0)
        def _(out_h=out_h):
            pltpu.sync_copy(acc_s.at[pl.ds(NS * RPT, REM)],
                            out_h.at[c, pl.ds(NS * RPT, REM)])

        plsc.subcore_barrier()


_agg_kernel = pl.kernel(
    _agg_body,
    out_type=[
        jax.ShapeDtypeStruct((NC, N, D), jnp.float32),
        jax.ShapeDtypeStruct((NC, N, D), jnp.float32),
        jax.ShapeDtypeStruct((NC, N, D), jnp.float32),
    ],
    mesh=_mesh,
    scratch_types=(
        [pltpu.VMEM((HCPT, CHUNK), jnp.int32),
         pltpu.VMEM((HCPT, CHUNK), jnp.int32),
         pltpu.VMEM((HCPT, CHUNK), jnp.float32)]
        + [pltpu.VMEM((CHUNK, D), jnp.float32)] * NBUF
        + [pltpu.SemaphoreType.DMA] * (2 * NBUF)
        + [pltpu.MemorySpace.VMEM_SHARED((N, D), jnp.float32)]
    ),
)


def _final_body(a1_ref, a2_ref, a3_ref, dinv_ref, b1_ref, b2_ref, b3_ref,
                fc_ref, fcb_ref, cw_ref, out_ref):
    dinv = dinv_ref[...]
    ys = []
    for p, (a_ref, b_ref) in enumerate(((a1_ref, b1_ref), (a2_ref, b2_ref),
                                        (a3_ref, b3_ref))):
        a = a_ref[0] + a_ref[1]
        y = jnp.maximum(dinv[p] * a + b_ref[...], 0.0)
        ys.append(y)
    t = jnp.dot(ys[0], fc_ref[0:D, :], preferred_element_type=jnp.float32)
    t = t + cw_ref[0] * jnp.dot(ys[1], fc_ref[D:2 * D, :],
                                preferred_element_type=jnp.float32)
    t = t + cw_ref[1] * jnp.dot(ys[2], fc_ref[2 * D:3 * D, :],
                                preferred_element_type=jnp.float32)
    out_ref[...] = jnp.maximum(t + fcb_ref[...], 0.0)


def _final(acc1, acc2, acc3, dinv, b1, b2, b3, fck, fcb, cw):
    grid = N // _BLK
    return pl.pallas_call(
        _final_body,
        grid=(grid,),
        in_specs=[
            pl.BlockSpec((NC, _BLK, D), lambda i: (0, i, 0)),
            pl.BlockSpec((NC, _BLK, D), lambda i: (0, i, 0)),
            pl.BlockSpec((NC, _BLK, D), lambda i: (0, i, 0)),
            pl.BlockSpec((3, _BLK, 1), lambda i: (0, i, 0)),
            pl.BlockSpec((1, D), lambda i: (0, 0)),
            pl.BlockSpec((1, D), lambda i: (0, 0)),
            pl.BlockSpec((1, D), lambda i: (0, 0)),
            pl.BlockSpec((3 * D, D), lambda i: (0, 0)),
            pl.BlockSpec((1, D), lambda i: (0, 0)),
            pl.BlockSpec(memory_space=pltpu.MemorySpace.SMEM),
        ],
        out_specs=pl.BlockSpec((_BLK, D), lambda i: (i, 0)),
        out_shape=jax.ShapeDtypeStruct((N, D), jnp.float32),
    )(acc1, acc2, acc3, dinv, b1.reshape(1, D), b2.reshape(1, D),
      b3.reshape(1, D), fck, fcb.reshape(1, D), cw)


def _pad_edges(edge_index, edge_weight):
    row = edge_index[0].astype(jnp.int32)
    col = edge_index[1].astype(jnp.int32)
    w = edge_weight.astype(jnp.float32)
    pad = EP - E
    pidx = jnp.arange(pad, dtype=jnp.int32) % N
    row2d = jnp.concatenate([row, pidx]).reshape(ROWS, CHUNK)
    col2d = jnp.concatenate([col, pidx]).reshape(ROWS, CHUNK)
    w2d = jnp.concatenate([w, jnp.zeros((pad,), jnp.float32)]
                          ).reshape(ROWS, CHUNK)
    return row2d, col2d, w2d


def kernel(x, edge_index_1st_prox, edge_weight_1st_prox,
           edge_index_2nd_prox_in, edge_weight_2nd_prox_in,
           edge_index_2nd_prox_out, edge_weight_2nd_prox_out,
           kernel1, kernel2in, kernel2out, bias1, bias2in, bias2out,
           concate_weight2in, concate_weight2out, fc_kernel, fc_bias):
    row1, col1, w1 = _pad_edges(edge_index_1st_prox, edge_weight_1st_prox)
    row2, col2, w2 = _pad_edges(edge_index_2nd_prox_in,
                                edge_weight_2nd_prox_in)
    row3, col3, w3 = _pad_edges(edge_index_2nd_prox_out,
                                edge_weight_2nd_prox_out)

    zn = jnp.zeros((N,), jnp.float32)
    zf = jnp.zeros((N, D), jnp.float32)

    d10, d20, d30, d11, d21, d31 = _deg_kernel(row1, w1, row2, w2,
                                               row3, w3, zn)
    deg_part = jnp.stack([jnp.stack([d10, d20, d30]),
                          jnp.stack([d11, d21, d31])])
    h1, h2, h3, dinv = _proj(x, kernel1, kernel2in, kernel2out, deg_part)
    acc1, acc2, acc3 = _agg_kernel(h1, col1, row1, w1, h2, col2, row2, w2,
                                   h3, col3, row3, w3, zf)
    cw = jnp.concatenate([concate_weight2in, concate_weight2out])
    return _final(acc1, acc2, acc3, dinv, bias1, bias2in, bias2out,
                  fc_kernel, fc_bias, cw)
